# Initial kernel scaffold; baseline (speedup 1.0000x reference)
import jax
import jax.numpy as jnp
from jax.experimental import pallas as pl


def kernel(x, adj_t, edge_weight, W1, b1, W2, b2):
    raise NotImplementedError("write your pallas kernel here")


# speedup vs baseline: 12.4428x; 12.4428x over previous
import functools

import jax
import jax.numpy as jnp
from jax import lax
from jax.experimental import pallas as pl
from jax.experimental.pallas import tpu as pltpu
from jax.experimental.pallas import tpu_sc as plsc

_NC = 2
_NS = 16
_NW = _NC * _NS
_L = 16
_CH = 128


def _pad_edges(n_edges):
    ept = -(-n_edges // (_NW * _CH)) * _CH
    return ept, ept // _CH, ept * _NW


def _make_deg_kernel(n_nodes, n_edges):
    ept, nch, _ = _pad_edges(n_edges)
    mesh = plsc.VectorSubcoreMesh(core_axis_name="c", subcore_axis_name="s")

    @functools.partial(
        pl.kernel,
        out_type=jax.ShapeDtypeStruct((_NC, n_nodes), jnp.float32),
        mesh=mesh,
        scratch_types=[
            pltpu.VMEM((_CH,), jnp.int32),
            pltpu.VMEM((_CH,), jnp.float32),
            pltpu.VMEM_SHARED((n_nodes,), jnp.float32),
        ],
        compiler_params=pltpu.CompilerParams(needs_layout_passes=False, use_tc_tiling_on_sc=False),
    )
    def deg_kernel(col_hbm, ew_hbm, zeros_hbm, out_hbm, idx_v, ew_v, acc_sh):
        c = lax.axis_index("c")
        s = lax.axis_index("s")
        wid = c * _NS + s

        @pl.when(s == 0)
        def _zero():
            pltpu.sync_copy(zeros_hbm, acc_sh)

        plsc.subcore_barrier()

        base = wid * ept

        def body(i, carry):
            off = base + i * _CH
            pltpu.sync_copy(col_hbm.at[pl.ds(off, _CH)], idx_v)
            pltpu.sync_copy(ew_hbm.at[pl.ds(off, _CH)], ew_v)
            pltpu.sync_copy(ew_v, acc_sh.at[idx_v], add=True)
            return carry

        lax.fori_loop(0, nch, body, 0)
        plsc.subcore_barrier()

        @pl.when(s == 0)
        def _dump():
            pltpu.sync_copy(acc_sh, out_hbm.at[c])

    return deg_kernel


def _make_agg_kernel(n_nodes, n_edges, feat):
    ept, nch, _ = _pad_edges(n_edges)
    rpt = (n_nodes // _NS) & ~7
    rlast = n_nodes - rpt * (_NS - 1)
    mesh = plsc.VectorSubcoreMesh(core_axis_name="c", subcore_axis_name="s")

    @functools.partial(
        pl.kernel,
        out_type=jax.ShapeDtypeStruct((_NC, n_nodes, feat), jnp.float32),
        mesh=mesh,
        scratch_types=[
            pltpu.VMEM((_CH,), jnp.int32),
            pltpu.VMEM((_CH,), jnp.int32),
            pltpu.VMEM((_CH,), jnp.float32),
            pltpu.VMEM((_CH, feat), jnp.float32),
            pltpu.VMEM_SHARED((n_nodes, feat), jnp.float32),
            pltpu.SemaphoreType.DMA,
        ],
        compiler_params=pltpu.CompilerParams(needs_layout_passes=False, use_tc_tiling_on_sc=False),
    )
    def agg_kernel(y_hbm, row_hbm, col_hbm, ew_hbm, zeros_hbm, out_hbm,
                   idxr_v, idxc_v, ew_v, rows_v, acc_sh, sem):
        c = lax.axis_index("c")
        s = lax.axis_index("s")
        wid = c * _NS + s

        @pl.when(s < _NS - 1)
        def _zero_main():
            stripe = pl.ds(s * rpt, rpt)
            pltpu.sync_copy(zeros_hbm.at[stripe], acc_sh.at[stripe])

        @pl.when(s == _NS - 1)
        def _zero_last():
            stripe = pl.ds((_NS - 1) * rpt, rlast)
            pltpu.sync_copy(zeros_hbm.at[stripe], acc_sh.at[stripe])

        plsc.subcore_barrier()

        base = wid * ept

        def body(i, carry):
            off = base + i * _CH
            pltpu.sync_copy(row_hbm.at[pl.ds(off, _CH)], idxr_v)
            pltpu.sync_copy(col_hbm.at[pl.ds(off, _CH)], idxc_v)
            pltpu.sync_copy(ew_hbm.at[pl.ds(off, _CH)], ew_v)
            pltpu.async_copy(y_hbm.at[idxr_v], rows_v, sem).wait()

            def scale(e, cc):
                w = plsc.load_gather(ew_v, [jnp.zeros((_L,), jnp.int32) + e])
                for f in range(feat // _L):
                    sl = pl.ds(f * _L, _L)
                    rows_v[e, sl] = rows_v[e, sl] * w
                return cc

            lax.fori_loop(0, _CH, scale, 0)
            pltpu.sync_copy(rows_v, acc_sh.at[idxc_v], add=True)
            return carry

        lax.fori_loop(0, nch, body, 0)
        plsc.subcore_barrier()

        @pl.when(s < _NS - 1)
        def _dump_main():
            stripe = pl.ds(s * rpt, rpt)
            pltpu.sync_copy(acc_sh.at[stripe], out_hbm.at[c].at[stripe])

        @pl.when(s == _NS - 1)
        def _dump_last():
            stripe = pl.ds((_NS - 1) * rpt, rlast)
            pltpu.sync_copy(acc_sh.at[stripe], out_hbm.at[c].at[stripe])

    return agg_kernel


def _matmul_tc(x, w):
    m, k = x.shape
    n = w.shape[1]

    def body(x_ref, w_ref, o_ref):
        o_ref[...] = jnp.dot(x_ref[...], w_ref[...],
                             preferred_element_type=jnp.float32)

    return pl.pallas_call(
        body,
        out_shape=jax.ShapeDtypeStruct((m, n), jnp.float32),
    )(x, w)


def _norm_tc(degp, xl):
    n, f = xl.shape

    def body(degp_ref, xl_ref, dis_ref, y_ref):
        d = 1.0 + degp_ref[0, :] + degp_ref[1, :]
        dis = 1.0 / jnp.sqrt(d)
        dis_ref[...] = dis[:, None]
        y_ref[...] = xl_ref[...] * dis[:, None]

    return pl.pallas_call(
        body,
        out_shape=(
            jax.ShapeDtypeStruct((n, 1), jnp.float32),
            jax.ShapeDtypeStruct((n, f), jnp.float32),
        ),
    )(degp, xl)


def _layer_mid_tc(p1, y1, dis, b1, w2p):
    n = y1.shape[0]
    f2 = w2p.shape[1]

    def body(p_ref, y1_ref, dis_ref, b1_ref, w2_ref, y2_ref):
        t = p_ref[0] + p_ref[1] + y1_ref[...]
        h = jnp.maximum(t * dis_ref[...] + b1_ref[...], 0.0)
        xl2 = jnp.dot(h, w2_ref[...], preferred_element_type=jnp.float32)
        y2_ref[...] = xl2 * dis_ref[...]

    return pl.pallas_call(
        body,
        out_shape=jax.ShapeDtypeStruct((n, f2), jnp.float32),
    )(p1, y1, dis, b1, w2p)


def _final_tc(p2, y2, dis, b2, out_ch):
    n = y2.shape[0]

    def body(p_ref, y2_ref, dis_ref, b2_ref, o_ref):
        t = p_ref[0] + p_ref[1] + y2_ref[...]
        o = t[:, :out_ch] * dis_ref[...] + b2_ref[...]
        m = jnp.max(o, axis=1, keepdims=True)
        e = jnp.exp(o - m)
        o_ref[...] = o - m - jnp.log(jnp.sum(e, axis=1, keepdims=True))

    return pl.pallas_call(
        body,
        out_shape=jax.ShapeDtypeStruct((n, out_ch), jnp.float32),
    )(p2, y2, dis, b2)


def kernel(x, adj_t, edge_weight, W1, b1, W2, b2):
    n = x.shape[0]
    n_edges = edge_weight.shape[0]
    hid = W1.shape[1]
    out_ch = W2.shape[1]
    f2 = -(-out_ch // _L) * _L

    _, _, e_pad = _pad_edges(n_edges)
    pad = e_pad - n_edges
    row = jnp.concatenate(
        [adj_t[0].astype(jnp.int32), jnp.zeros((pad,), jnp.int32)])
    col = jnp.concatenate(
        [adj_t[1].astype(jnp.int32), jnp.zeros((pad,), jnp.int32)])
    ew = jnp.concatenate(
        [edge_weight, jnp.zeros((pad,), jnp.float32)])

    zeros1 = jnp.zeros((n,), jnp.float32)
    zeros_h = jnp.zeros((n, hid), jnp.float32)
    zeros_f2 = jnp.zeros((n, f2), jnp.float32)
    w2p = jnp.concatenate(
        [W2, jnp.zeros((hid, f2 - out_ch), jnp.float32)], axis=1)

    degp = _make_deg_kernel(n, n_edges)(col, ew, zeros1)
    xl1 = _matmul_tc(x, W1)
    dis, y1 = _norm_tc(degp, xl1)
    p1 = _make_agg_kernel(n, n_edges, hid)(y1, row, col, ew, zeros_h)
    y2 = _layer_mid_tc(p1, y1, dis, b1.reshape(1, hid), w2p)
    p2 = _make_agg_kernel(n, n_edges, f2)(y2, row, col, ew, zeros_f2)
    return _final_tc(p2, y2, dis, b2.reshape(1, out_ch), out_ch)


# speedup vs baseline: 29.7919x; 2.3943x over previous
import functools

import jax
import jax.numpy as jnp
from jax import lax
from jax.experimental import pallas as pl
from jax.experimental.pallas import tpu as pltpu
from jax.experimental.pallas import tpu_sc as plsc

_NC = 2
_NS = 16
_NW = _NC * _NS
_L = 16
_CH = 128
_NBUF = 3

_SC_PARAMS = pltpu.CompilerParams(
    needs_layout_passes=False, use_tc_tiling_on_sc=False)


def _pad_edges(n_edges):
    ept = -(-n_edges // (_NW * _CH)) * _CH
    return ept, ept // _CH, ept * _NW


def _make_deg_kernel(n_nodes, n_edges):
    _, nch, _ = _pad_edges(n_edges)
    mesh = plsc.VectorSubcoreMesh(core_axis_name="c", subcore_axis_name="s")

    @functools.partial(
        pl.kernel,
        out_type=jax.ShapeDtypeStruct((_NC, n_nodes), jnp.float32),
        mesh=mesh,
        scratch_types=[
            pltpu.VMEM((nch, _CH), jnp.int32),
            pltpu.VMEM((nch, _CH), jnp.float32),
            pltpu.VMEM_SHARED((n_nodes,), jnp.float32),
            pltpu.SemaphoreType.DMA,
        ],
        compiler_params=_SC_PARAMS,
    )
    def deg_kernel(col_hbm, ew_hbm, zeros_hbm, out_hbm, col_v, ew_v, acc_sh,
                   sem):
        c = lax.axis_index("c")
        s = lax.axis_index("s")
        wid = c * _NS + s

        @pl.when(s == 0)
        def _zero():
            pltpu.sync_copy(zeros_hbm, acc_sh)

        pltpu.sync_copy(col_hbm.at[wid], col_v)
        pltpu.sync_copy(ew_hbm.at[wid], ew_v)
        plsc.subcore_barrier()

        def body(i, carry):
            pltpu.async_copy(ew_v.at[i], acc_sh.at[col_v.at[i]], sem,
                             add=True)
            return carry

        lax.fori_loop(0, nch, body, 0)

        def drain(i, carry):
            pltpu.make_async_copy(ew_v.at[i], acc_sh.at[col_v.at[i]],
                                  sem).wait()
            return carry

        lax.fori_loop(0, nch, drain, 0)
        plsc.subcore_barrier()

        @pl.when(s == 0)
        def _dump():
            pltpu.sync_copy(acc_sh, out_hbm.at[c])

    return deg_kernel


def _make_agg_kernel(n_nodes, n_edges, feat):
    _, nch, _ = _pad_edges(n_edges)
    rpt = (n_nodes // _NS) & ~7
    rlast = n_nodes - rpt * (_NS - 1)
    mesh = plsc.VectorSubcoreMesh(core_axis_name="c", subcore_axis_name="s")

    @functools.partial(
        pl.kernel,
        out_type=jax.ShapeDtypeStruct((_NC, n_nodes, feat), jnp.float32),
        mesh=mesh,
        scratch_types=[
            pltpu.VMEM((nch, _CH), jnp.int32),
            pltpu.VMEM((nch, _CH), jnp.int32),
            pltpu.VMEM((nch, _CH), jnp.float32),
            [pltpu.VMEM((_CH, feat), jnp.float32)] * _NBUF,
            pltpu.VMEM_SHARED((n_nodes, feat), jnp.float32),
            [pltpu.SemaphoreType.DMA] * _NBUF,
            [pltpu.SemaphoreType.DMA] * _NBUF,
        ],
        compiler_params=_SC_PARAMS,
    )
    def agg_kernel(y_hbm, row_hbm, col_hbm, ew_hbm, zeros_hbm, out_hbm,
                   row_v, col_v, ew_v, rows_bufs, acc_sh, gsems, ssems):
        c = lax.axis_index("c")
        s = lax.axis_index("s")
        wid = c * _NS + s

        @pl.when(s < _NS - 1)
        def _zero_main():
            stripe = pl.ds(s * rpt, rpt)
            pltpu.sync_copy(zeros_hbm.at[stripe], acc_sh.at[stripe])

        @pl.when(s == _NS - 1)
        def _zero_last():
            stripe = pl.ds((_NS - 1) * rpt, rlast)
            pltpu.sync_copy(zeros_hbm.at[stripe], acc_sh.at[stripe])

        pltpu.sync_copy(row_hbm.at[wid], row_v)
        pltpu.sync_copy(col_hbm.at[wid], col_v)
        pltpu.sync_copy(ew_hbm.at[wid], ew_v)
        plsc.subcore_barrier()

        def start_gather(i, b):
            pltpu.async_copy(y_hbm.at[row_v.at[i]], rows_bufs[b], gsems[b])

        def wait_gather(i, b):
            pltpu.make_async_copy(y_hbm.at[row_v.at[i]], rows_bufs[b],
                                  gsems[b]).wait()

        def start_scatter(i, b):
            pltpu.async_copy(rows_bufs[b], acc_sh.at[col_v.at[i]], ssems[b],
                             add=True)

        def wait_scatter(i, b):
            pltpu.make_async_copy(rows_bufs[b], acc_sh.at[col_v.at[i]],
                                  ssems[b]).wait()

        def scale_chunk(b, i):
            buf = rows_bufs[b]

            def scale_edge(e, carry):
                w = plsc.load_gather(
                    ew_v, [jnp.zeros((_L,), jnp.int32) + i,
                           jnp.zeros((_L,), jnp.int32) + e])
                for f in range(feat // _L):
                    sl = pl.ds(f * _L, _L)
                    buf[e, sl] = buf[e, sl] * w
                return carry

            lax.fori_loop(0, _CH, scale_edge, 0, unroll=4)

        start_gather(0, 0)
        nsteps = -(-nch // _NBUF)

        def body(i0, carry):
            for b in range(_NBUF):
                i = i0 * _NBUF + b

                @pl.when(i < nch)
                def _step():
                    bn = (b + 1) % _NBUF

                    @pl.when(i + 1 < nch)
                    def _prefetch():
                        @pl.when(i + 1 >= _NBUF)
                        def _free():
                            wait_scatter(i + 1 - _NBUF, bn)

                        start_gather(i + 1, bn)

                    wait_gather(i, b)
                    scale_chunk(b, i)
                    start_scatter(i, b)

            return carry

        lax.fori_loop(0, nsteps, body, 0)

        for k in range(min(_NBUF, nch)):
            i = nch - 1 - k
            wait_scatter(i, i % _NBUF)

        plsc.subcore_barrier()

        @pl.when(s < _NS - 1)
        def _dump_main():
            stripe = pl.ds(s * rpt, rpt)
            pltpu.sync_copy(acc_sh.at[stripe], out_hbm.at[c].at[stripe])

        @pl.when(s == _NS - 1)
        def _dump_last():
            stripe = pl.ds((_NS - 1) * rpt, rlast)
            pltpu.sync_copy(acc_sh.at[stripe], out_hbm.at[c].at[stripe])

    return agg_kernel


def _matmul_tc(x, w):
    m, k = x.shape
    n = w.shape[1]

    def body(x_ref, w_ref, o_ref):
        o_ref[...] = jnp.dot(x_ref[...], w_ref[...],
                             preferred_element_type=jnp.float32)

    return pl.pallas_call(
        body,
        out_shape=jax.ShapeDtypeStruct((m, n), jnp.float32),
    )(x, w)


def _norm_tc(degp, xl):
    n, f = xl.shape

    def body(degp_ref, xl_ref, dis_ref, y_ref):
        d = 1.0 + degp_ref[0, :] + degp_ref[1, :]
        dis = 1.0 / jnp.sqrt(d)
        dis_ref[...] = dis[:, None]
        y_ref[...] = xl_ref[...] * dis[:, None]

    return pl.pallas_call(
        body,
        out_shape=(
            jax.ShapeDtypeStruct((n, 1), jnp.float32),
            jax.ShapeDtypeStruct((n, f), jnp.float32),
        ),
    )(degp, xl)


def _layer_mid_tc(p1, y1, dis, b1, w2p):
    n = y1.shape[0]
    f2 = w2p.shape[1]

    def body(p_ref, y1_ref, dis_ref, b1_ref, w2_ref, y2_ref):
        t = p_ref[0] + p_ref[1] + y1_ref[...]
        h = jnp.maximum(t * dis_ref[...] + b1_ref[...], 0.0)
        xl2 = jnp.dot(h, w2_ref[...], preferred_element_type=jnp.float32)
        y2_ref[...] = xl2 * dis_ref[...]

    return pl.pallas_call(
        body,
        out_shape=jax.ShapeDtypeStruct((n, f2), jnp.float32),
    )(p1, y1, dis, b1, w2p)


def _final_tc(p2, y2, dis, b2, out_ch):
    n = y2.shape[0]

    def body(p_ref, y2_ref, dis_ref, b2_ref, o_ref):
        t = p_ref[0] + p_ref[1] + y2_ref[...]
        o = t[:, :out_ch] * dis_ref[...] + b2_ref[...]
        m = jnp.max(o, axis=1, keepdims=True)
        e = jnp.exp(o - m)
        o_ref[...] = o - m - jnp.log(jnp.sum(e, axis=1, keepdims=True))

    return pl.pallas_call(
        body,
        out_shape=jax.ShapeDtypeStruct((n, out_ch), jnp.float32),
    )(p2, y2, dis, b2)


def kernel(x, adj_t, edge_weight, W1, b1, W2, b2):
    n = x.shape[0]
    n_edges = edge_weight.shape[0]
    hid = W1.shape[1]
    out_ch = W2.shape[1]
    f2 = -(-out_ch // _L) * _L

    _, nch, e_pad = _pad_edges(n_edges)
    pad = e_pad - n_edges
    row = jnp.concatenate(
        [adj_t[0].astype(jnp.int32), jnp.zeros((pad,), jnp.int32)])
    col = jnp.concatenate(
        [adj_t[1].astype(jnp.int32), jnp.zeros((pad,), jnp.int32)])
    ew = jnp.concatenate(
        [edge_weight, jnp.zeros((pad,), jnp.float32)])
    row = row.reshape(_NW, nch, _CH)
    col = col.reshape(_NW, nch, _CH)
    ew = ew.reshape(_NW, nch, _CH)

    zeros1 = jnp.zeros((n,), jnp.float32)
    zeros_h = jnp.zeros((n, hid), jnp.float32)
    zeros_f2 = jnp.zeros((n, f2), jnp.float32)
    w2p = jnp.concatenate(
        [W2, jnp.zeros((hid, f2 - out_ch), jnp.float32)], axis=1)

    degp = _make_deg_kernel(n, n_edges)(col, ew, zeros1)
    xl1 = _matmul_tc(x, W1)
    dis, y1 = _norm_tc(degp, xl1)
    p1 = _make_agg_kernel(n, n_edges, hid)(y1, row, col, ew, zeros_h)
    y2 = _layer_mid_tc(p1, y1, dis, b1.reshape(1, hid), w2p)
    p2 = _make_agg_kernel(n, n_edges, f2)(y2, row, col, ew, zeros_f2)
    return _final_tc(p2, y2, dis, b2.reshape(1, out_ch), out_ch)


# speedup vs baseline: 38.7644x; 1.3012x over previous
import functools

import jax
import jax.numpy as jnp
from jax import lax
from jax.experimental import pallas as pl
from jax.experimental.pallas import tpu as pltpu
from jax.experimental.pallas import tpu_sc as plsc

_NC = 2
_NS = 16
_NW = _NC * _NS
_L = 16
_CH = 128
_SPS = 2
_NBUF = 4

_SC_PARAMS = pltpu.CompilerParams(
    needs_layout_passes=False, use_tc_tiling_on_sc=False)


def _pad_edges(n_edges):
    ept = -(-n_edges // (_NW * _SPS * _CH)) * _SPS * _CH
    return ept, ept // _CH, ept * _NW


def _make_deg_kernel(n_nodes, n_edges):
    _, nch, _ = _pad_edges(n_edges)
    mesh = plsc.VectorSubcoreMesh(core_axis_name="c", subcore_axis_name="s")

    @functools.partial(
        pl.kernel,
        out_type=jax.ShapeDtypeStruct((_NC, n_nodes), jnp.float32),
        mesh=mesh,
        scratch_types=[
            pltpu.VMEM((nch, _CH), jnp.int32),
            pltpu.VMEM((nch, _CH), jnp.float32),
            pltpu.VMEM_SHARED((n_nodes,), jnp.float32),
            pltpu.SemaphoreType.DMA,
        ],
        compiler_params=_SC_PARAMS,
    )
    def deg_kernel(col_hbm, ew_hbm, zeros_hbm, out_hbm, col_v, ew_v, acc_sh,
                   sem):
        c = lax.axis_index("c")
        s = lax.axis_index("s")
        wid = c * _NS + s

        @pl.when(s == 0)
        def _zero():
            pltpu.sync_copy(zeros_hbm, acc_sh)

        pltpu.sync_copy(col_hbm.at[wid], col_v)
        pltpu.sync_copy(ew_hbm.at[wid], ew_v)
        plsc.subcore_barrier()

        def body(i, carry):
            pltpu.async_copy(ew_v.at[i], acc_sh.at[col_v.at[i]], sem,
                             add=True)
            return carry

        lax.fori_loop(0, nch, body, 0)

        def drain(i, carry):
            pltpu.make_async_copy(ew_v.at[i], acc_sh.at[col_v.at[i]],
                                  sem).wait()
            return carry

        lax.fori_loop(0, nch, drain, 0)
        plsc.subcore_barrier()

        @pl.when(s == 0)
        def _dump():
            pltpu.sync_copy(acc_sh, out_hbm.at[c])

    return deg_kernel


def _make_agg_kernel(n_nodes, n_edges, feat):
    _, nch, _ = _pad_edges(n_edges)
    rpt = (n_nodes // _NS) & ~7
    rlast = n_nodes - rpt * (_NS - 1)
    mesh = plsc.VectorSubcoreMesh(core_axis_name="c", subcore_axis_name="s")

    @functools.partial(
        pl.kernel,
        out_type=jax.ShapeDtypeStruct((_NC, n_nodes, feat), jnp.float32),
        mesh=mesh,
        scratch_types=[
            pltpu.VMEM((nch, _CH), jnp.int32),
            pltpu.VMEM((nch, _CH), jnp.int32),
            pltpu.VMEM((nch // _SPS, _SPS * _CH), jnp.float32),
            [pltpu.VMEM((_SPS * _CH, feat), jnp.float32)] * _NBUF,
            pltpu.VMEM_SHARED((n_nodes, feat), jnp.float32),
            [pltpu.SemaphoreType.DMA] * _NBUF,
            [pltpu.SemaphoreType.DMA] * _NBUF,
        ],
        compiler_params=_SC_PARAMS,
    )
    def agg_kernel(y_hbm, row_hbm, col_hbm, ew_hbm, zeros_hbm, out_hbm,
                   row_v, col_v, ew_v, rows_bufs, acc_sh, gsems, ssems):
        c = lax.axis_index("c")
        s = lax.axis_index("s")
        wid = c * _NS + s

        @pl.when(s < _NS - 1)
        def _zero_main():
            stripe = pl.ds(s * rpt, rpt)
            pltpu.sync_copy(zeros_hbm.at[stripe], acc_sh.at[stripe])

        @pl.when(s == _NS - 1)
        def _zero_last():
            stripe = pl.ds((_NS - 1) * rpt, rlast)
            pltpu.sync_copy(zeros_hbm.at[stripe], acc_sh.at[stripe])

        pltpu.sync_copy(row_hbm.at[wid], row_v)
        pltpu.sync_copy(col_hbm.at[wid], col_v)
        pltpu.sync_copy(ew_hbm.at[wid], ew_v)
        plsc.subcore_barrier()

        nslot = nch // _SPS

        def start_gather(p, b):
            buf = rows_bufs[b]
            for k in range(_SPS):
                pltpu.async_copy(y_hbm.at[row_v.at[_SPS * p + k]],
                                 buf.at[pl.ds(k * _CH, _CH)], gsems[b])

        def wait_gather(p, b):
            buf = rows_bufs[b]
            for k in range(_SPS):
                pltpu.make_async_copy(y_hbm.at[row_v.at[_SPS * p + k]],
                                      buf.at[pl.ds(k * _CH, _CH)],
                                      gsems[b]).wait()

        def start_scatter(p, b):
            buf = rows_bufs[b]
            for k in range(_SPS):
                pltpu.async_copy(buf.at[pl.ds(k * _CH, _CH)],
                                 acc_sh.at[col_v.at[_SPS * p + k]],
                                 ssems[b], add=True)

        def wait_scatter(p, b):
            buf = rows_bufs[b]
            for k in range(_SPS):
                pltpu.make_async_copy(buf.at[pl.ds(k * _CH, _CH)],
                                      acc_sh.at[col_v.at[_SPS * p + k]],
                                      ssems[b]).wait()

        def scale_slot(b, p):
            buf = rows_bufs[b]
            zp = jnp.zeros((_L,), jnp.int32) + p

            def scale_edge(e, carry):
                w = plsc.load_gather(
                    ew_v, [zp, jnp.zeros((_L,), jnp.int32) + e])
                for f in range(feat // _L):
                    sl = pl.ds(f * _L, _L)
                    buf[e, sl] = buf[e, sl] * w
                return carry

            lax.fori_loop(0, _SPS * _CH, scale_edge, 0, unroll=8)

        start_gather(0, 0)
        nsteps = -(-nslot // _NBUF)

        def body(p0, carry):
            for b in range(_NBUF):
                p = p0 * _NBUF + b

                @pl.when(p < nslot)
                def _step():
                    bn = (b + 1) % _NBUF

                    @pl.when(p + 1 < nslot)
                    def _prefetch():
                        @pl.when(p + 1 >= _NBUF)
                        def _free():
                            wait_scatter(p + 1 - _NBUF, bn)

                        start_gather(p + 1, bn)

                    wait_gather(p, b)
                    scale_slot(b, p)
                    start_scatter(p, b)

            return carry

        lax.fori_loop(0, nsteps, body, 0)

        for k in range(min(_NBUF, nslot)):
            p = nslot - 1 - k
            wait_scatter(p, p % _NBUF)

        plsc.subcore_barrier()

        @pl.when(s < _NS - 1)
        def _dump_main():
            stripe = pl.ds(s * rpt, rpt)
            pltpu.sync_copy(acc_sh.at[stripe], out_hbm.at[c].at[stripe])

        @pl.when(s == _NS - 1)
        def _dump_last():
            stripe = pl.ds((_NS - 1) * rpt, rlast)
            pltpu.sync_copy(acc_sh.at[stripe], out_hbm.at[c].at[stripe])

    return agg_kernel


def _mm_norm_tc(x, w, degp):
    n, f = x.shape[0], w.shape[1]

    def body(x_ref, w_ref, degp_ref, dis_ref, y_ref):
        xl = jnp.dot(x_ref[...], w_ref[...],
                     preferred_element_type=jnp.float32)
        d = 1.0 + degp_ref[0, :] + degp_ref[1, :]
        dis = 1.0 / jnp.sqrt(d)
        dis_ref[...] = dis[:, None]
        y_ref[...] = xl * dis[:, None]

    return pl.pallas_call(
        body,
        out_shape=(
            jax.ShapeDtypeStruct((n, 1), jnp.float32),
            jax.ShapeDtypeStruct((n, f), jnp.float32),
        ),
    )(x, w, degp)


def _layer_mid_tc(p1, y1, dis, b1, w2p):
    n = y1.shape[0]
    f2 = w2p.shape[1]

    def body(p_ref, y1_ref, dis_ref, b1_ref, w2_ref, y2_ref):
        t = p_ref[0] + p_ref[1] + y1_ref[...]
        h = jnp.maximum(t * dis_ref[...] + b1_ref[...], 0.0)
        xl2 = jnp.dot(h, w2_ref[...], preferred_element_type=jnp.float32)
        y2_ref[...] = xl2 * dis_ref[...]

    return pl.pallas_call(
        body,
        out_shape=jax.ShapeDtypeStruct((n, f2), jnp.float32),
    )(p1, y1, dis, b1, w2p)


def _final_tc(p2, y2, dis, b2, out_ch):
    n = y2.shape[0]

    def body(p_ref, y2_ref, dis_ref, b2_ref, o_ref):
        t = p_ref[0] + p_ref[1] + y2_ref[...]
        o = t[:, :out_ch] * dis_ref[...] + b2_ref[...]
        m = jnp.max(o, axis=1, keepdims=True)
        e = jnp.exp(o - m)
        o_ref[...] = o - m - jnp.log(jnp.sum(e, axis=1, keepdims=True))

    return pl.pallas_call(
        body,
        out_shape=jax.ShapeDtypeStruct((n, out_ch), jnp.float32),
    )(p2, y2, dis, b2)


def kernel(x, adj_t, edge_weight, W1, b1, W2, b2):
    n = x.shape[0]
    n_edges = edge_weight.shape[0]
    hid = W1.shape[1]
    out_ch = W2.shape[1]
    f2 = -(-out_ch // _L) * _L

    _, nch, e_pad = _pad_edges(n_edges)
    pad = e_pad - n_edges
    pad_idx = jnp.arange(pad, dtype=jnp.int32) % n
    row = jnp.concatenate([adj_t[0].astype(jnp.int32), pad_idx])
    col = jnp.concatenate([adj_t[1].astype(jnp.int32), pad_idx])
    ew = jnp.concatenate(
        [edge_weight, jnp.zeros((pad,), jnp.float32)])
    row = row.reshape(_NW, nch, _CH)
    col = col.reshape(_NW, nch, _CH)
    ewc = ew.reshape(_NW, nch, _CH)
    ewp = ew.reshape(_NW, nch // _SPS, _SPS * _CH)

    zeros1 = jnp.zeros((n,), jnp.float32)
    zeros_h = jnp.zeros((n, hid), jnp.float32)
    zeros_f2 = jnp.zeros((n, f2), jnp.float32)
    w2p = jnp.concatenate(
        [W2, jnp.zeros((hid, f2 - out_ch), jnp.float32)], axis=1)

    degp = _make_deg_kernel(n, n_edges)(col, ewc, zeros1)
    dis, y1 = _mm_norm_tc(x, W1, degp)
    p1 = _make_agg_kernel(n, n_edges, hid)(y1, row, col, ewp, zeros_h)
    y2 = _layer_mid_tc(p1, y1, dis, b1.reshape(1, hid), w2p)
    p2 = _make_agg_kernel(n, n_edges, f2)(y2, row, col, ewp, zeros_f2)
    return _final_tc(p2, y2, dis, b2.reshape(1, out_ch), out_ch)
